# baseline (speedup 1.0000x reference)
import numpy as np
import jax
import jax.numpy as jnp
from jax.experimental import pallas as pl
from jax.experimental.pallas import tpu as pltpu

_N_WIDTH = 256
_N_ORDER = 8
_N_ELEMENTS = 64
_N_NODES = _N_ELEMENTS * _N_ORDER + 1
_N_COLL = 100
_DELTA_X = 0.5 * _N_ORDER / (_N_NODES - 1)
_NODES = np.linspace(-1.0, 1.0, _N_ORDER + 1)
_NB = _N_ORDER + 1

_ROWS_PER_BLOCK = 10


def _inv_table():
    v = np.zeros((16, _NB), np.float32)
    for j in range(_NB):
        for m in range(_NB):
            if j != m:
                v[j, m] = 1.0 / (_NODES[j] - _NODES[m])
    return v


_INV_TABLE = _inv_table()


def _omit_one_products(fs):
    n = len(fs)
    pre = [None] * n
    suf = [None] * n
    acc = None
    for i in range(n):
        pre[i] = acc
        acc = fs[i] if acc is None else acc * fs[i]
    acc = None
    for i in range(n - 1, -1, -1):
        suf[i] = acc
        acc = fs[i] if acc is None else acc * fs[i]
    out = []
    for i in range(n):
        if pre[i] is None:
            out.append(suf[i])
        elif suf[i] is None:
            out.append(pre[i])
        else:
            out.append(pre[i] * suf[i])
    return out


def _basis_rows(x_t, inv_tab):
    j2 = jax.lax.broadcasted_iota(jnp.int32, (16, _N_WIDTH), 0)
    xb = jnp.broadcast_to(x_t, (16, _N_WIDTH)).astype(jnp.float32)
    f = []
    cs = [inv_tab[:, m : m + 1] for m in range(_NB)]
    for m in range(_NB):
        f.append(jnp.where(j2 == m, 1.0, (xb - np.float32(_NODES[m])) * cs[m]))

    phi = f[0]
    for m in range(1, _NB):
        phi = phi * f[m]

    p1 = _omit_one_products(f)
    dphi = cs[0] * p1[0]
    for i in range(1, _NB):
        dphi = dphi + cs[i] * p1[i]

    ddphi = None
    for i in range(_NB):
        idxs = [m for m in range(_NB) if m != i]
        q = _omit_one_products([f[m] for m in idxs])
        inner = None
        for pos, m in enumerate(idxs):
            term = cs[m] * q[pos]
            inner = term if inner is None else inner + term
        term = cs[i] * inner
        ddphi = term if ddphi is None else ddphi + term

    dphi = dphi * np.float32(1.0 / _DELTA_X)
    ddphi = ddphi * np.float32(1.0 / (_DELTA_X * _DELTA_X))
    return phi, dphi, ddphi


def _rows_body(x_ref, w_ref, inv_ref, phi_r, dphi_r, ddphi_r, t_r, dt_r, ddt_r):
    xs = x_ref[...]
    x_shift = (_N_NODES - 1.0) * xs
    id_el = jnp.clip(jnp.floor(x_shift / _N_ORDER), 0.0, _N_ELEMENTS - 1.0)
    nl_f = id_el * _N_ORDER
    x_t = (x_shift - nl_f - 0.5 * _N_ORDER) / (0.5 * _N_ORDER)
    phi, dphi, ddphi = _basis_rows(x_t, inv_ref[...])

    ident = (
        jax.lax.broadcasted_iota(jnp.int32, (_N_WIDTH, _N_WIDTH), 0)
        == jax.lax.broadcasted_iota(jnp.int32, (_N_WIDTH, _N_WIDTH), 1)
    ).astype(jnp.float32)

    def to_col(row):
        return jnp.sum(ident * row, axis=1, keepdims=True)

    def to_row(col):
        return jnp.sum(ident * col, axis=0, keepdims=True)

    nl_col = to_col(nl_f).astype(jnp.int32)
    rel = jax.lax.broadcasted_iota(jnp.int32, (_N_WIDTH, _N_NODES), 1) - nl_col

    bases = (phi, dphi, ddphi)
    dense = [jnp.zeros((_N_WIDTH, _N_NODES), jnp.float32) for _ in range(3)]
    for j in range(_NB):
        mask = rel == j
        for b in range(3):
            colv = to_col(bases[b][j : j + 1, :])
            dense[b] = jnp.where(mask, colv, dense[b])

    phi_r[...] = dense[0]
    dphi_r[...] = dense[1]
    ddphi_r[...] = dense[2]

    w = w_ref[...]
    for d, tr in zip(dense, (t_r, dt_r, ddt_r)):
        tcol = jnp.sum(w * d, axis=1, keepdims=True)
        tr[...] = to_row(tcol)


_rows_call = pl.pallas_call(
    _rows_body,
    out_shape=(
        [jax.ShapeDtypeStruct((_N_WIDTH, _N_NODES), jnp.float32)] * 3
        + [jax.ShapeDtypeStruct((1, _N_WIDTH), jnp.float32)] * 3
    ),
)


def _assemble_body(s_ref, rphi, rdphi, rddphi, phi_o, dphi_o, ddphi_o):
    i = pl.program_id(0)
    s = s_ref[0]
    base = i * _ROWS_PER_BLOCK
    z = jnp.zeros((_ROWS_PER_BLOCK, _N_WIDTH, _N_NODES), jnp.float32)
    phi_o[...] = z
    dphi_o[...] = z
    ddphi_o[...] = z

    @pl.when((s >= base) & (s < base + _ROWS_PER_BLOCK))
    def _():
        loc = s - base
        phi_o[pl.ds(loc, 1), :, :] = rphi[...][None]
        dphi_o[pl.ds(loc, 1), :, :] = rdphi[...][None]
        ddphi_o[pl.ds(loc, 1), :, :] = rddphi[...][None]


_assemble_call = pl.pallas_call(
    _assemble_body,
    grid=(_N_COLL // _ROWS_PER_BLOCK,),
    in_specs=[
        pl.BlockSpec(memory_space=pltpu.SMEM),
        pl.BlockSpec((_N_WIDTH, _N_NODES), lambda i: (0, 0)),
        pl.BlockSpec((_N_WIDTH, _N_NODES), lambda i: (0, 0)),
        pl.BlockSpec((_N_WIDTH, _N_NODES), lambda i: (0, 0)),
    ],
    out_specs=[
        pl.BlockSpec((_ROWS_PER_BLOCK, _N_WIDTH, _N_NODES), lambda i: (i, 0, 0)),
        pl.BlockSpec((_ROWS_PER_BLOCK, _N_WIDTH, _N_NODES), lambda i: (i, 0, 0)),
        pl.BlockSpec((_ROWS_PER_BLOCK, _N_WIDTH, _N_NODES), lambda i: (i, 0, 0)),
    ],
    out_shape=[
        jax.ShapeDtypeStruct((_N_COLL, _N_WIDTH, _N_NODES), jnp.float32)
    ] * 3,
)


def kernel(x, epoch, sample, weight):
    del epoch
    s = jnp.asarray(sample, jnp.int32).reshape((1,))
    rphi, rdphi, rddphi, t, dt, ddt = _rows_call(x, weight, jnp.asarray(_INV_TABLE))
    phi_buf, dphi_buf, ddphi_buf = _assemble_call(s, rphi, rdphi, rddphi)
    return (t, dt, ddt, phi_buf, dphi_buf, ddphi_buf, jnp.float32(_DELTA_X))


# speedup vs baseline: 1.0096x; 1.0096x over previous
import numpy as np
import jax
import jax.numpy as jnp
from jax.experimental import pallas as pl
from jax.experimental.pallas import tpu as pltpu

_N_WIDTH = 256
_N_ORDER = 8
_N_ELEMENTS = 64
_N_NODES = _N_ELEMENTS * _N_ORDER + 1
_N_COLL = 100
_DELTA_X = 0.5 * _N_ORDER / (_N_NODES - 1)
_NODES = np.linspace(-1.0, 1.0, _N_ORDER + 1)
_NB = _N_ORDER + 1

_BLK = 10


def _inv_table():
    v = np.zeros((16, _NB), np.float32)
    for j in range(_NB):
        for m in range(_NB):
            if j != m:
                v[j, m] = 1.0 / (_NODES[j] - _NODES[m])
    return v


_INV_TABLE = _inv_table()


def _omit_one_products(fs):
    n = len(fs)
    pre = [None] * n
    suf = [None] * n
    acc = None
    for i in range(n):
        pre[i] = acc
        acc = fs[i] if acc is None else acc * fs[i]
    acc = None
    for i in range(n - 1, -1, -1):
        suf[i] = acc
        acc = fs[i] if acc is None else acc * fs[i]
    out = []
    for i in range(n):
        if pre[i] is None:
            out.append(suf[i])
        elif suf[i] is None:
            out.append(pre[i])
        else:
            out.append(pre[i] * suf[i])
    return out


def _basis_rows(x_t, inv_tab):
    j2 = jax.lax.broadcasted_iota(jnp.int32, (16, _N_WIDTH), 0)
    xb = jnp.broadcast_to(x_t, (16, _N_WIDTH)).astype(jnp.float32)
    f = []
    cs = [inv_tab[:, m : m + 1] for m in range(_NB)]
    for m in range(_NB):
        f.append(jnp.where(j2 == m, 1.0, (xb - np.float32(_NODES[m])) * cs[m]))

    phi = f[0]
    for m in range(1, _NB):
        phi = phi * f[m]

    p1 = _omit_one_products(f)
    dphi = cs[0] * p1[0]
    for i in range(1, _NB):
        dphi = dphi + cs[i] * p1[i]

    ddphi = None
    for i in range(_NB):
        idxs = [m for m in range(_NB) if m != i]
        q = _omit_one_products([f[m] for m in idxs])
        inner = None
        for pos, m in enumerate(idxs):
            term = cs[m] * q[pos]
            inner = term if inner is None else inner + term
        term = cs[i] * inner
        ddphi = term if ddphi is None else ddphi + term

    dphi = dphi * np.float32(1.0 / _DELTA_X)
    ddphi = ddphi * np.float32(1.0 / (_DELTA_X * _DELTA_X))
    return phi, dphi, ddphi


def _fused_body(
    s_ref,
    x_ref,
    w_ref,
    inv_ref,
    t_r,
    dt_r,
    ddt_r,
    phi_o,
    dphi_o,
    ddphi_o,
    zblk,
    rb0,
    rb1,
    rb2,
    sem_z,
    sem_r,
):
    n_blocks = _N_COLL // _BLK
    outs = (phi_o, dphi_o, ddphi_o)

    zblk[...] = jnp.zeros((_BLK, _N_WIDTH, _N_NODES), jnp.float32)
    for i in range(n_blocks):
        for o in outs:
            pltpu.make_async_copy(
                zblk, o.at[pl.ds(i * _BLK, _BLK)], sem_z
            ).start()

    xs = x_ref[...]
    x_shift = (_N_NODES - 1.0) * xs
    id_el = jnp.clip(jnp.floor(x_shift / _N_ORDER), 0.0, _N_ELEMENTS - 1.0)
    nl_f = id_el * _N_ORDER
    x_t = (x_shift - nl_f - 0.5 * _N_ORDER) / (0.5 * _N_ORDER)
    phi, dphi, ddphi = _basis_rows(x_t, inv_ref[...])

    ident = (
        jax.lax.broadcasted_iota(jnp.int32, (_N_WIDTH, _N_WIDTH), 0)
        == jax.lax.broadcasted_iota(jnp.int32, (_N_WIDTH, _N_WIDTH), 1)
    ).astype(jnp.float32)

    def to_col(row):
        return jnp.sum(ident * row, axis=1, keepdims=True)

    def to_row(col):
        return jnp.sum(ident * col, axis=0, keepdims=True)

    nl_col = to_col(nl_f).astype(jnp.int32)
    rel = jax.lax.broadcasted_iota(jnp.int32, (_N_WIDTH, _N_NODES), 1) - nl_col

    bases = (phi, dphi, ddphi)
    dense = [jnp.zeros((_N_WIDTH, _N_NODES), jnp.float32) for _ in range(3)]
    for j in range(_NB):
        mask = rel == j
        for b in range(3):
            colv = to_col(bases[b][j : j + 1, :])
            dense[b] = jnp.where(mask, colv, dense[b])

    rbs = (rb0, rb1, rb2)
    for rb, d in zip(rbs, dense):
        rb[...] = d[None]

    w = w_ref[...]
    for d, tr in zip(dense, (t_r, dt_r, ddt_r)):
        tcol = jnp.sum(w * d, axis=1, keepdims=True)
        tr[...] = to_row(tcol)

    for i in range(n_blocks):
        for o in outs:
            pltpu.make_async_copy(
                zblk, o.at[pl.ds(i * _BLK, _BLK)], sem_z
            ).wait()

    s = s_ref[0]
    for rb, o in zip(rbs, outs):
        pltpu.make_async_copy(rb, o.at[pl.ds(s, 1)], sem_r).start()
    for rb, o in zip(rbs, outs):
        pltpu.make_async_copy(rb, o.at[pl.ds(s, 1)], sem_r).wait()


_big = jax.ShapeDtypeStruct((_N_COLL, _N_WIDTH, _N_NODES), jnp.float32)
_vec = jax.ShapeDtypeStruct((1, _N_WIDTH), jnp.float32)

_fused_call = pl.pallas_call(
    _fused_body,
    in_specs=[
        pl.BlockSpec(memory_space=pltpu.SMEM),
        pl.BlockSpec((1, _N_WIDTH), lambda: (0, 0)),
        pl.BlockSpec((_N_WIDTH, _N_NODES), lambda: (0, 0)),
        pl.BlockSpec((16, _NB), lambda: (0, 0)),
    ],
    out_specs=[
        pl.BlockSpec((1, _N_WIDTH), lambda: (0, 0)),
        pl.BlockSpec((1, _N_WIDTH), lambda: (0, 0)),
        pl.BlockSpec((1, _N_WIDTH), lambda: (0, 0)),
        pl.BlockSpec(memory_space=pltpu.MemorySpace.HBM),
        pl.BlockSpec(memory_space=pltpu.MemorySpace.HBM),
        pl.BlockSpec(memory_space=pltpu.MemorySpace.HBM),
    ],
    out_shape=[_vec, _vec, _vec, _big, _big, _big],
    scratch_shapes=[
        pltpu.VMEM((_BLK, _N_WIDTH, _N_NODES), jnp.float32),
        pltpu.VMEM((1, _N_WIDTH, _N_NODES), jnp.float32),
        pltpu.VMEM((1, _N_WIDTH, _N_NODES), jnp.float32),
        pltpu.VMEM((1, _N_WIDTH, _N_NODES), jnp.float32),
        pltpu.SemaphoreType.DMA,
        pltpu.SemaphoreType.DMA,
    ],
)


def kernel(x, epoch, sample, weight):
    del epoch
    s = jnp.asarray(sample, jnp.int32).reshape((1,))
    t, dt, ddt, phi_buf, dphi_buf, ddphi_buf = _fused_call(
        s, x, weight, jnp.asarray(_INV_TABLE)
    )
    return (t, dt, ddt, phi_buf, dphi_buf, ddphi_buf, jnp.float32(_DELTA_X))
